# Initial kernel scaffold; baseline (speedup 1.0000x reference)
import jax
import jax.numpy as jnp
from jax.experimental import pallas as pl


def kernel(x, edge_index, W1, b1, g1, be1, W2, b2, g2, be2, Wc1, bc1, Wc2, bc2):
    raise NotImplementedError("write your pallas kernel here")


# speedup vs baseline: 3.4069x; 3.4069x over previous
import functools

import jax
import jax.numpy as jnp
from jax import lax
from jax.experimental import pallas as pl
from jax.experimental.pallas import tpu as pltpu
from jax.experimental.pallas import tpu_sc as plsc

N = 10000
NP = 10112
E = 320000
EP = 327680
H = 64
T1W = 80
T2W = 64

NC, NS = 2, 16
NW = NC * NS
ROWS_PER_TILE_SC = NP // NS
IDX_ROWS = EP // 128
GRP = 8

_LEAK = 0.2


def _make_sc_round1():
  mesh = plsc.VectorSubcoreMesh(
      core_axis_name="c", subcore_axis_name="s", num_cores=NC, num_subcores=NS)
  rows_per_tile = IDX_ROWS // NS
  groups = rows_per_tile // GRP

  @functools.partial(
      pl.kernel,
      out_type=jax.ShapeDtypeStruct((NC, NP, T1W), jnp.float32),
      mesh=mesh,
      compiler_params=pltpu.CompilerParams(use_tc_tiling_on_sc=False),
      scratch_types=[
          pltpu.VMEM((GRP, 128), jnp.int32),
          pltpu.VMEM((GRP, 128), jnp.int32),
          pltpu.VMEM((128, T1W), jnp.float32),
          pltpu.VMEM((ROWS_PER_TILE_SC, T1W), jnp.float32),
          pltpu.VMEM_SHARED((NP, T1W), jnp.float32),
          pltpu.SemaphoreType.DMA,
      ],
  )
  def sc_round1(table_hbm, src_hbm, dst_hbm, zeros_hbm, out_hbm,
                sidx_v, didx_v, rows_v, stage_v, acc_sh, sem):
    c = lax.axis_index("c")
    s = lax.axis_index("s")

    pltpu.sync_copy(zeros_hbm, stage_v)
    pltpu.sync_copy(
        stage_v, acc_sh.at[pl.ds(s * ROWS_PER_TILE_SC, ROWS_PER_TILE_SC)])
    plsc.subcore_barrier()

    base_row = s * rows_per_tile

    def group_body(g, carry):
      r0 = base_row + g * GRP
      pltpu.sync_copy(src_hbm.at[pl.ds(r0, GRP)], sidx_v)
      pltpu.sync_copy(dst_hbm.at[pl.ds(r0, GRP)], didx_v)
      for j in range(GRP):
        pltpu.async_copy(
            table_hbm.at[c].at[sidx_v.at[j]], rows_v, sem).wait()
        pltpu.sync_copy(rows_v, acc_sh.at[didx_v.at[j]], add=True)
      return carry

    lax.fori_loop(0, groups, group_body, 0)
    plsc.subcore_barrier()

    pltpu.sync_copy(
        acc_sh.at[pl.ds(s * ROWS_PER_TILE_SC, ROWS_PER_TILE_SC)], stage_v)
    pltpu.sync_copy(
        stage_v,
        out_hbm.at[c, pl.ds(s * ROWS_PER_TILE_SC, ROWS_PER_TILE_SC), :])

  return sc_round1


def _make_sc_round2():
  mesh = plsc.VectorSubcoreMesh(
      core_axis_name="c", subcore_axis_name="s", num_cores=NC, num_subcores=NS)
  rows_per_w = IDX_ROWS // NW
  groups = rows_per_w // GRP

  @functools.partial(
      pl.kernel,
      out_type=jax.ShapeDtypeStruct((NC, NP, T2W), jnp.float32),
      mesh=mesh,
      compiler_params=pltpu.CompilerParams(use_tc_tiling_on_sc=False),
      scratch_types=[
          pltpu.VMEM((GRP, 128), jnp.int32),
          pltpu.VMEM((GRP, 128), jnp.int32),
          pltpu.VMEM((128, T2W), jnp.float32),
          pltpu.VMEM((ROWS_PER_TILE_SC, T2W), jnp.float32),
          pltpu.VMEM_SHARED((NP, T2W), jnp.float32),
          pltpu.SemaphoreType.DMA,
      ],
  )
  def sc_round2(table_hbm, src_hbm, dst_hbm, zeros_hbm, out_hbm,
                sidx_v, didx_v, rows_v, stage_v, acc_sh, sem):
    c = lax.axis_index("c")
    s = lax.axis_index("s")
    w = s * NC + c

    pltpu.sync_copy(zeros_hbm, stage_v)
    pltpu.sync_copy(
        stage_v, acc_sh.at[pl.ds(s * ROWS_PER_TILE_SC, ROWS_PER_TILE_SC)])
    plsc.subcore_barrier()

    base_row = w * rows_per_w

    def group_body(g, carry):
      r0 = base_row + g * GRP
      pltpu.sync_copy(src_hbm.at[pl.ds(r0, GRP)], sidx_v)
      pltpu.sync_copy(dst_hbm.at[pl.ds(r0, GRP)], didx_v)
      for j in range(GRP):
        pltpu.async_copy(table_hbm.at[sidx_v.at[j]], rows_v, sem).wait()
        pltpu.sync_copy(rows_v, acc_sh.at[didx_v.at[j]], add=True)
      return carry

    lax.fori_loop(0, groups, group_body, 0)
    plsc.subcore_barrier()

    pltpu.sync_copy(
        acc_sh.at[pl.ds(s * ROWS_PER_TILE_SC, ROWS_PER_TILE_SC)], stage_v)
    pltpu.sync_copy(
        stage_v,
        out_hbm.at[c, pl.ds(s * ROWS_PER_TILE_SC, ROWS_PER_TILE_SC), :])

  return sc_round2


_sc_cache = {}


def _sc_round(which):
  if which not in _sc_cache:
    _sc_cache[which] = _make_sc_round1() if which == 1 else _make_sc_round2()
  return _sc_cache[which]


def _leaky(x):
  return jnp.where(x >= 0, x, _LEAK * x)


def _bf16_dot(a, b):
  return jnp.dot(a.astype(jnp.bfloat16), b.astype(jnp.bfloat16),
                 preferred_element_type=jnp.float32)


def _row_mask():
  rows = lax.broadcasted_iota(jnp.int32, (NP, 1), 0)
  return (rows < N).astype(jnp.float32)


def _bn_leaky(h, g, b):
  mask = _row_mask()
  hm = h * mask
  m = jnp.sum(hm, axis=0, keepdims=True) * (1.0 / N)
  v = jnp.sum(hm * hm, axis=0, keepdims=True) * (1.0 / N) - m * m
  hb = g[None, :] * (h - m) * lax.rsqrt(v + 1e-5) + b[None, :]
  return _leaky(hb)


def _tc_combine1(x_pad, agg1, b1, g1, be1, w1t):
  def body(x_ref, agg_ref, b1_ref, g1_ref, be1_ref, w1t_ref, o_ref):
    agg = jnp.concatenate([agg_ref[0, :, 0:H], agg_ref[1, :, 0:H]], axis=1)
    cnt = agg_ref[0, :, H:H + 1]
    rden = 1.0 / (1.0 + jnp.maximum(cnt, 1.0))
    t = (x_ref[...] + agg) * rden
    h = _leaky(_bf16_dot(t, w1t_ref[...]) + b1_ref[...][None, :])
    hb = _bn_leaky(h, g1_ref[...], be1_ref[...])
    o_ref[...] = hb * _row_mask()

  return pl.pallas_call(
      body, out_shape=jax.ShapeDtypeStruct((NP, T2W), jnp.float32))(
          x_pad, agg1, b1, g1, be1, w1t)


def _tc_combine2(table2, acc2, agg1, b2, g2, be2, w2t, wc1t, bc1, wc2t, bc2):
  def body(t2_ref, acc2_ref, agg1_ref, b2_ref, g2_ref, be2_ref, w2t_ref,
           wc1t_ref, bc1_ref, wc2t_ref, bc2_ref, o_ref):
    agg = acc2_ref[0] + acc2_ref[1]
    cnt = agg1_ref[0, :, H:H + 1]
    rden = 1.0 / (1.0 + jnp.maximum(cnt, 1.0))
    t = (t2_ref[...] + agg) * rden
    h = _leaky(_bf16_dot(t, w2t_ref[...]) + b2_ref[...][None, :])
    hb = _bn_leaky(h, g2_ref[...], be2_ref[...])
    hc = jnp.maximum(_bf16_dot(hb, wc1t_ref[...]) + bc1_ref[...][None, :], 0.0)
    logits = _bf16_dot(hc, wc2t_ref[...])
    o_ref[...] = logits + bc2_ref[...][None, :]

  return pl.pallas_call(
      body, out_shape=jax.ShapeDtypeStruct((NP, 128), jnp.float32))(
          table2, acc2, agg1, b2, g2, be2, w2t, wc1t, bc1, wc2t, bc2)


def kernel(x, edge_index, W1, b1, g1, be1, W2, b2, g2, be2, Wc1, bc1, Wc2, bc2):
  x_pad = jnp.zeros((NP, 128), jnp.float32).at[:N].set(x)
  ones_col = jnp.zeros((NP, T1W - H), jnp.float32).at[:N, 0].set(1.0)
  table1 = jnp.stack([
      jnp.concatenate([x_pad[:, 0:H], ones_col], axis=1),
      jnp.concatenate([x_pad[:, H:128], jnp.zeros((NP, T1W - H), jnp.float32)],
                      axis=1),
  ])
  src = edge_index[0]
  dst = edge_index[1]
  pad = jnp.full((EP - E,), N, jnp.int32)
  src2d = jnp.concatenate([src, pad]).reshape(IDX_ROWS, 128)
  dst2d = jnp.concatenate([dst, pad]).reshape(IDX_ROWS, 128)
  zeros1 = jnp.zeros((ROWS_PER_TILE_SC, T1W), jnp.float32)
  zeros2 = jnp.zeros((ROWS_PER_TILE_SC, T2W), jnp.float32)
  w1t = W1.T
  w2t = W2.T
  wc1t = jnp.zeros((H, 128), jnp.float32).at[:, :32].set(Wc1.T)
  bc1p = jnp.zeros((128,), jnp.float32).at[:32].set(bc1)
  wc2t = jnp.zeros((128, 128), jnp.float32).at[:32, :2].set(Wc2.T)
  bc2p = jnp.zeros((128,), jnp.float32).at[:2].set(bc2)

  agg1 = _sc_round(1)(table1, src2d, dst2d, zeros1)
  table2 = _tc_combine1(x_pad, agg1, b1, g1, be1, w1t)

  acc2 = _sc_round(2)(table2, src2d, dst2d, zeros2)
  out = _tc_combine2(table2, acc2, agg1, b2, g2, be2, w2t, wc1t, bc1p,
                     wc2t, bc2p)

  return out[:N, :2]


# speedup vs baseline: 3.7769x; 1.1086x over previous
import functools

import jax
import jax.numpy as jnp
from jax import lax
from jax.experimental import pallas as pl
from jax.experimental.pallas import tpu as pltpu
from jax.experimental.pallas import tpu_sc as plsc

N = 10000
NP = 10112
E = 320000
EP = 327680
H = 64
T1W = 80
T2W = 64

NC, NS = 2, 16
NW = NC * NS
ROWS_PER_TILE_SC = NP // NS
IDX_ROWS = EP // 128
GRP = 8

_LEAK = 0.2


def _make_sc_round(dw, feature_split):
  mesh = plsc.VectorSubcoreMesh(
      core_axis_name="c", subcore_axis_name="s", num_cores=NC, num_subcores=NS)
  phase_rows = 80
  phases = 2 if feature_split else 1
  n_groups = phase_rows // 2

  @functools.partial(
      pl.kernel,
      out_type=jax.ShapeDtypeStruct((NC, NP, dw), jnp.float32),
      mesh=mesh,
      compiler_params=pltpu.CompilerParams(use_tc_tiling_on_sc=False),
      scratch_types=[
          pltpu.VMEM((phase_rows, 128), jnp.int32),
          pltpu.VMEM((phase_rows, 128), jnp.int32),
          pltpu.VMEM((4, 128, dw), jnp.float32),
          pltpu.VMEM_SHARED((NP, dw), jnp.float32),
          pltpu.SemaphoreType.DMA,
          pltpu.SemaphoreType.DMA,
      ],
  )
  def sc_round(table_hbm, src_hbm, dst_hbm, zeros_hbm, out_hbm,
               sidx_v, didx_v, bufs_v, acc_sh, sem_g, sem_s):
    c = lax.axis_index("c")
    s = lax.axis_index("s")

    pltpu.sync_copy(
        zeros_hbm, acc_sh.at[pl.ds(s * ROWS_PER_TILE_SC, ROWS_PER_TILE_SC)])
    plsc.subcore_barrier()

    tbl = table_hbm.at[c] if feature_split else table_hbm

    def start_g(g, bank):
      for k in range(2):
        pltpu.async_copy(
            tbl.at[sidx_v.at[2 * g + k]], bufs_v.at[2 * bank + k], sem_g)

    def drain_g(bank):
      for k in range(2):
        pltpu.make_async_copy(
            tbl.at[pl.ds(0, 128)], bufs_v.at[2 * bank + k], sem_g).wait()

    def start_s(g, bank):
      for k in range(2):
        pltpu.sync_copy(
            bufs_v.at[2 * bank + k], acc_sh.at[didx_v.at[2 * g + k]],
            add=True)

    def drain_s(bank):
      del bank

    for h in range(phases):
      if feature_split:
        base = s * (phases * phase_rows) + h * phase_rows
      else:
        base = (s * NC + c) * phase_rows
      pltpu.sync_copy(src_hbm.at[pl.ds(base, phase_rows)], sidx_v)
      pltpu.sync_copy(dst_hbm.at[pl.ds(base, phase_rows)], didx_v)

      start_g(0, 0)
      drain_g(0)
      start_s(0, 0)
      start_g(1, 1)

      def pair_body(p, carry):
        for b in (1, 0):
          g = 2 * p + (1 if b == 1 else 2)
          drain_g(b)
          start_s(g, b)
          drain_s(1 - b)
          start_g(g + 1, 1 - b)
        return carry

      lax.fori_loop(0, (n_groups - 2) // 2, pair_body, 0)

      drain_g(1)
      start_s(n_groups - 1, 1)
      drain_s(0)
      drain_s(1)

    plsc.subcore_barrier()
    pltpu.sync_copy(
        acc_sh.at[pl.ds(s * ROWS_PER_TILE_SC, ROWS_PER_TILE_SC)],
        out_hbm.at[c, pl.ds(s * ROWS_PER_TILE_SC, ROWS_PER_TILE_SC), :])

  return sc_round


_sc_cache = {}


def _sc_round(which):
  if which not in _sc_cache:
    _sc_cache[which] = (_make_sc_round(T1W, True) if which == 1
                        else _make_sc_round(T2W, False))
  return _sc_cache[which]


def _leaky(x):
  return jnp.where(x >= 0, x, _LEAK * x)


def _bf16_dot(a, b):
  return jnp.dot(a.astype(jnp.bfloat16), b.astype(jnp.bfloat16),
                 preferred_element_type=jnp.float32)


def _row_mask():
  rows = lax.broadcasted_iota(jnp.int32, (NP, 1), 0)
  return (rows < N).astype(jnp.float32)


def _bn_leaky(h, g, b):
  mask = _row_mask()
  hm = h * mask
  m = jnp.sum(hm, axis=0, keepdims=True) * (1.0 / N)
  v = jnp.sum(hm * hm, axis=0, keepdims=True) * (1.0 / N) - m * m
  hb = g[None, :] * (h - m) * lax.rsqrt(v + 1e-5) + b[None, :]
  return _leaky(hb)


def _tc_combine1(x_pad, agg1, b1, g1, be1, w1t):
  def body(x_ref, agg_ref, b1_ref, g1_ref, be1_ref, w1t_ref, o_ref):
    agg = jnp.concatenate([agg_ref[0, :, 0:H], agg_ref[1, :, 0:H]], axis=1)
    cnt = agg_ref[0, :, H:H + 1]
    rden = 1.0 / (1.0 + jnp.maximum(cnt, 1.0))
    t = (x_ref[...] + agg) * rden
    h = _leaky(_bf16_dot(t, w1t_ref[...]) + b1_ref[...][None, :])
    hb = _bn_leaky(h, g1_ref[...], be1_ref[...])
    o_ref[...] = hb * _row_mask()

  return pl.pallas_call(
      body, out_shape=jax.ShapeDtypeStruct((NP, T2W), jnp.float32))(
          x_pad, agg1, b1, g1, be1, w1t)


def _tc_combine2(table2, acc2, agg1, b2, g2, be2, w2t, wc1t, bc1, wc2t, bc2):
  def body(t2_ref, acc2_ref, agg1_ref, b2_ref, g2_ref, be2_ref, w2t_ref,
           wc1t_ref, bc1_ref, wc2t_ref, bc2_ref, o_ref):
    agg = acc2_ref[0] + acc2_ref[1]
    cnt = agg1_ref[0, :, H:H + 1]
    rden = 1.0 / (1.0 + jnp.maximum(cnt, 1.0))
    t = (t2_ref[...] + agg) * rden
    h = _leaky(_bf16_dot(t, w2t_ref[...]) + b2_ref[...][None, :])
    hb = _bn_leaky(h, g2_ref[...], be2_ref[...])
    hc = jnp.maximum(_bf16_dot(hb, wc1t_ref[...]) + bc1_ref[...][None, :], 0.0)
    logits = _bf16_dot(hc, wc2t_ref[...])
    o_ref[...] = logits + bc2_ref[...][None, :]

  return pl.pallas_call(
      body, out_shape=jax.ShapeDtypeStruct((NP, 128), jnp.float32))(
          table2, acc2, agg1, b2, g2, be2, w2t, wc1t, bc1, wc2t, bc2)


def kernel(x, edge_index, W1, b1, g1, be1, W2, b2, g2, be2, Wc1, bc1, Wc2, bc2):
  x_pad = jnp.zeros((NP, 128), jnp.float32).at[:N].set(x)
  ones_col = jnp.zeros((NP, T1W - H), jnp.float32).at[:N, 0].set(1.0)
  table1 = jnp.stack([
      jnp.concatenate([x_pad[:, 0:H], ones_col], axis=1),
      jnp.concatenate([x_pad[:, H:128], jnp.zeros((NP, T1W - H), jnp.float32)],
                      axis=1),
  ])
  src = edge_index[0]
  dst = edge_index[1]
  pad = jnp.full((EP - E,), N, jnp.int32)
  src2d = jnp.concatenate([src, pad]).reshape(IDX_ROWS, 128)
  dst2d = jnp.concatenate([dst, pad]).reshape(IDX_ROWS, 128)
  zeros1 = jnp.zeros((ROWS_PER_TILE_SC, T1W), jnp.float32)
  zeros2 = jnp.zeros((ROWS_PER_TILE_SC, T2W), jnp.float32)
  w1t = W1.T
  w2t = W2.T
  wc1t = jnp.zeros((H, 128), jnp.float32).at[:, :32].set(Wc1.T)
  bc1p = jnp.zeros((128,), jnp.float32).at[:32].set(bc1)
  wc2t = jnp.zeros((128, 128), jnp.float32).at[:32, :2].set(Wc2.T)
  bc2p = jnp.zeros((128,), jnp.float32).at[:2].set(bc2)

  agg1 = _sc_round(1)(table1, src2d, dst2d, zeros1)
  table2 = _tc_combine1(x_pad, agg1, b1, g1, be1, w1t)

  acc2 = _sc_round(2)(table2, src2d, dst2d, zeros2)
  out = _tc_combine2(table2, acc2, agg1, b2, g2, be2, w2t, wc1t, bc1p,
                     wc2t, bc2p)

  return out[:N, :2]


# speedup vs baseline: 4.1621x; 1.1020x over previous
import functools

import jax
import jax.numpy as jnp
from jax import lax
from jax.experimental import pallas as pl
from jax.experimental.pallas import tpu as pltpu
from jax.experimental.pallas import tpu_sc as plsc

N = 10000
NP = 10112
E = 320000
EP = 327680
H = 64
T1W = 80
T2W = 64

NC, NS = 2, 16
NW = NC * NS
ROWS_PER_TILE_SC = NP // NS
IDX_ROWS = EP // 128
GRP = 8

_LEAK = 0.2


def _make_sc_round(dw, feature_split):
  mesh = plsc.VectorSubcoreMesh(
      core_axis_name="c", subcore_axis_name="s", num_cores=NC, num_subcores=NS)
  phase_rows = 80
  phases = 2 if feature_split else 1
  bank = 2
  n_groups = phase_rows // bank

  @functools.partial(
      pl.kernel,
      out_type=jax.ShapeDtypeStruct((NC, NP, dw), jnp.float32),
      mesh=mesh,
      compiler_params=pltpu.CompilerParams(use_tc_tiling_on_sc=False),
      scratch_types=[
          pltpu.VMEM((phase_rows, 128), jnp.int32),
          pltpu.VMEM((phase_rows, 128), jnp.int32),
          pltpu.VMEM((2 * bank, 128, dw), jnp.float32),
          pltpu.VMEM_SHARED((NP, dw), jnp.float32),
          pltpu.SemaphoreType.DMA,
          pltpu.SemaphoreType.DMA,
      ],
  )
  def sc_round(table_hbm, src_hbm, dst_hbm, zeros_hbm, out_hbm,
               sidx_v, didx_v, bufs_v, acc_sh, sem_g, sem_s):
    c = lax.axis_index("c")
    s = lax.axis_index("s")

    pltpu.sync_copy(
        zeros_hbm, acc_sh.at[pl.ds(s * ROWS_PER_TILE_SC, ROWS_PER_TILE_SC)])
    plsc.subcore_barrier()

    tbl = table_hbm.at[c] if feature_split else table_hbm

    def start_g(g, bank):
      for k in range(2):
        pltpu.async_copy(
            tbl.at[sidx_v.at[2 * g + k]], bufs_v.at[2 * bank + k], sem_g)

    def drain_g(bank):
      for k in range(2):
        pltpu.make_async_copy(
            tbl.at[pl.ds(0, 128)], bufs_v.at[2 * bank + k], sem_g).wait()

    def start_s(g, bank):
      for k in range(2):
        pltpu.async_copy(
            bufs_v.at[2 * bank + k], acc_sh.at[didx_v.at[2 * g + k]], sem_s,
            add=True)

    def drain_s(bank):
      for k in range(2):
        pltpu.make_async_copy(
            zeros_hbm.at[pl.ds(0, 128)], bufs_v.at[2 * bank + k], sem_s).wait()

    for h in range(phases):
      if feature_split:
        base = s * (phases * phase_rows) + h * phase_rows
      else:
        base = (s * NC + c) * phase_rows
      pltpu.sync_copy(src_hbm.at[pl.ds(base, phase_rows)], sidx_v)
      pltpu.sync_copy(dst_hbm.at[pl.ds(base, phase_rows)], didx_v)

      start_g(0, 0)
      drain_g(0)
      start_s(0, 0)
      start_g(1, 1)

      def pair_body(p, carry):
        for b in (1, 0):
          g = 2 * p + (1 if b == 1 else 2)
          drain_g(b)
          start_s(g, b)
          drain_s(1 - b)
          start_g(g + 1, 1 - b)
        return carry

      lax.fori_loop(0, (n_groups - 2) // 2, pair_body, 0)

      drain_g(1)
      start_s(n_groups - 1, 1)
      drain_s(0)
      drain_s(1)

    plsc.subcore_barrier()
    pltpu.sync_copy(
        acc_sh.at[pl.ds(s * ROWS_PER_TILE_SC, ROWS_PER_TILE_SC)],
        out_hbm.at[c, pl.ds(s * ROWS_PER_TILE_SC, ROWS_PER_TILE_SC), :])

  return sc_round


_sc_cache = {}


def _sc_round(which):
  if which not in _sc_cache:
    _sc_cache[which] = (_make_sc_round(T1W, True) if which == 1
                        else _make_sc_round(T2W, False))
  return _sc_cache[which]


def _leaky(x):
  return jnp.where(x >= 0, x, _LEAK * x)


def _bf16_dot(a, b):
  return jnp.dot(a.astype(jnp.bfloat16), b.astype(jnp.bfloat16),
                 preferred_element_type=jnp.float32)


def _row_mask():
  rows = lax.broadcasted_iota(jnp.int32, (NP, 1), 0)
  return (rows < N).astype(jnp.float32)


def _bn_leaky(h, g, b):
  mask = _row_mask()
  hm = h * mask
  m = jnp.sum(hm, axis=0, keepdims=True) * (1.0 / N)
  v = jnp.sum(hm * hm, axis=0, keepdims=True) * (1.0 / N) - m * m
  hb = g[None, :] * (h - m) * lax.rsqrt(v + 1e-5) + b[None, :]
  return _leaky(hb)


def _tc_combine1(x_pad, agg1, b1, g1, be1, w1t):
  def body(x_ref, agg_ref, b1_ref, g1_ref, be1_ref, w1t_ref, o_ref):
    agg = jnp.concatenate([agg_ref[0, :, 0:H], agg_ref[1, :, 0:H]], axis=1)
    cnt = agg_ref[0, :, H:H + 1]
    rden = 1.0 / (1.0 + jnp.maximum(cnt, 1.0))
    t = (x_ref[...] + agg) * rden
    h = _leaky(_bf16_dot(t, w1t_ref[...]) + b1_ref[...][None, :])
    hb = _bn_leaky(h, g1_ref[...], be1_ref[...])
    o_ref[...] = hb * _row_mask()

  return pl.pallas_call(
      body, out_shape=jax.ShapeDtypeStruct((NP, T2W), jnp.float32))(
          x_pad, agg1, b1, g1, be1, w1t)


def _tc_combine2(table2, acc2, agg1, b2, g2, be2, w2t, wc1t, bc1, wc2t, bc2):
  def body(t2_ref, acc2_ref, agg1_ref, b2_ref, g2_ref, be2_ref, w2t_ref,
           wc1t_ref, bc1_ref, wc2t_ref, bc2_ref, o_ref):
    agg = acc2_ref[0] + acc2_ref[1]
    cnt = agg1_ref[0, :, H:H + 1]
    rden = 1.0 / (1.0 + jnp.maximum(cnt, 1.0))
    t = (t2_ref[...] + agg) * rden
    h = _leaky(_bf16_dot(t, w2t_ref[...]) + b2_ref[...][None, :])
    hb = _bn_leaky(h, g2_ref[...], be2_ref[...])
    hc = jnp.maximum(_bf16_dot(hb, wc1t_ref[...]) + bc1_ref[...][None, :], 0.0)
    logits = _bf16_dot(hc, wc2t_ref[...])
    o_ref[...] = logits + bc2_ref[...][None, :]

  return pl.pallas_call(
      body, out_shape=jax.ShapeDtypeStruct((NP, 128), jnp.float32))(
          table2, acc2, agg1, b2, g2, be2, w2t, wc1t, bc1, wc2t, bc2)


def kernel(x, edge_index, W1, b1, g1, be1, W2, b2, g2, be2, Wc1, bc1, Wc2, bc2):
  x_pad = jnp.zeros((NP, 128), jnp.float32).at[:N].set(x)
  ones_col = jnp.zeros((NP, T1W - H), jnp.float32).at[:N, 0].set(1.0)
  table1 = jnp.stack([
      jnp.concatenate([x_pad[:, 0:H], ones_col], axis=1),
      jnp.concatenate([x_pad[:, H:128], jnp.zeros((NP, T1W - H), jnp.float32)],
                      axis=1),
  ])
  src = edge_index[0]
  dst = edge_index[1]
  pad = jnp.full((EP - E,), N, jnp.int32)
  src2d = jnp.concatenate([src, pad]).reshape(IDX_ROWS, 128)
  dst2d = jnp.concatenate([dst, pad]).reshape(IDX_ROWS, 128)
  zeros1 = jnp.zeros((ROWS_PER_TILE_SC, T1W), jnp.float32)
  zeros2 = jnp.zeros((ROWS_PER_TILE_SC, T2W), jnp.float32)
  w1t = W1.T
  w2t = W2.T
  wc1t = jnp.zeros((H, 128), jnp.float32).at[:, :32].set(Wc1.T)
  bc1p = jnp.zeros((128,), jnp.float32).at[:32].set(bc1)
  wc2t = jnp.zeros((128, 128), jnp.float32).at[:32, :2].set(Wc2.T)
  bc2p = jnp.zeros((128,), jnp.float32).at[:2].set(bc2)

  agg1 = _sc_round(1)(table1, src2d, dst2d, zeros1)
  table2 = _tc_combine1(x_pad, agg1, b1, g1, be1, w1t)

  acc2 = _sc_round(2)(table2, src2d, dst2d, zeros2)
  out = _tc_combine2(table2, acc2, agg1, b2, g2, be2, w2t, wc1t, bc1p,
                     wc2t, bc2p)

  return out[:N, :2]
